# baseline (speedup 1.0000x reference)
import jax
import jax.numpy as jnp
from jax import lax
from jax.experimental import pallas as pl
from jax.experimental.pallas import tpu as pltpu
from jax.experimental.pallas import tpu_sc as plsc

_N = 10000
_E = 5000
_NNZ = 320000
_D = 128
_HID = 128
_T = 16

_K = 128
_NCHUNK = _NNZ // _K
_NC = 2
_NS = 16
_NW = _NC * _NS
_ZROWS = 200


def _sc_mesh():
    return plsc.VectorSubcoreMesh(core_axis_name="c", subcore_axis_name="s")


def _sc_scatter_pass(r_out):
    nz = r_out // _ZROWS

    def body(in_hbm, si_hbm, di_hbm, out_hbm, si_v, di_v, rows_v, zero_v,
             acc, sem):
        c = lax.axis_index("c")
        s = lax.axis_index("s")
        w = s * _NC + c

        z16 = jnp.zeros((16,), jnp.float32)

        def zrow(i, carry):
            for j in range(128 // 16):
                zero_v[i, pl.ds(j * 16, 16)] = z16
            return carry

        lax.fori_loop(0, _ZROWS, zrow, 0)

        def zchunk(i, carry):
            t = s + i * _NS
            pltpu.sync_copy(zero_v, acc.at[pl.ds(t * _ZROWS, _ZROWS)])
            return carry

        lax.fori_loop(0, (nz - s + _NS - 1) // _NS, zchunk, 0)
        plsc.subcore_barrier()

        def chunk(i, carry):
            t = w + i * _NW
            base = t * _K
            pltpu.sync_copy(si_hbm.at[pl.ds(base, _K)], si_v)
            pltpu.sync_copy(di_hbm.at[pl.ds(base, _K)], di_v)
            pltpu.async_copy(in_hbm.at[si_v], rows_v, sem).wait()
            pltpu.sync_copy(rows_v, acc.at[di_v], add=True)
            return carry

        lax.fori_loop(0, (_NCHUNK - w + _NW - 1) // _NW, chunk, 0)
        plsc.subcore_barrier()

        def ochunk(i, carry):
            t = s + i * _NS
            pltpu.sync_copy(acc.at[pl.ds(t * _ZROWS, _ZROWS)],
                            out_hbm.at[c, pl.ds(t * _ZROWS, _ZROWS)])
            return carry

        lax.fori_loop(0, (nz - s + _NS - 1) // _NS, ochunk, 0)

    return pl.kernel(
        body,
        out_type=jax.ShapeDtypeStruct((_NC, r_out, 128), jnp.float32),
        mesh=_sc_mesh(),
        scratch_types=[
            pltpu.VMEM((_K,), jnp.int32),
            pltpu.VMEM((_K,), jnp.int32),
            pltpu.VMEM((_K, 128), jnp.float32),
            pltpu.VMEM((_ZROWS, 128), jnp.float32),
            pltpu.VMEM_SHARED((r_out, 128), jnp.float32),
            pltpu.SemaphoreType.DMA,
        ],
    )


def _tc_prep(vt2, wt, bt2, th1t, b12, cntn, cnte):
    def body(vt_ref, wt_ref, bt_ref, tht_ref, b1_ref, cn_ref, ce_ref,
             c1_ref, se_ref, ie_ref, idg_ref):
        vt = vt_ref[...]
        vmax = jnp.max(vt)
        vmean = jnp.sum(vt) * (1.0 / _E)
        tf = vmean * wt_ref[...] + bt_ref[...]
        c1_ref[...] = (jnp.dot(tf, tht_ref[...],
                               preferred_element_type=jnp.float32)
                       + b1_ref[...])
        decay = jnp.exp((vt - vmax) * (1.0 / 365.0))
        cn = cn_ref[...]
        ce = ce_ref[...]
        esize = jnp.maximum(ce[0] + ce[1], 1.0)
        ndeg = jnp.maximum(cn[0] + cn[1], 1.0)
        se_ref[...] = decay / esize
        ie_ref[...] = 1.0 / esize
        idg_ref[...] = 1.0 / ndeg

    return pl.pallas_call(
        body,
        out_shape=(jax.ShapeDtypeStruct((1, _HID), jnp.float32),
                   jax.ShapeDtypeStruct((_E, 1), jnp.float32),
                   jax.ShapeDtypeStruct((_E, 1), jnp.float32),
                   jax.ShapeDtypeStruct((_N, 1), jnp.float32)),
    )(vt2, wt, bt2, th1t, b12, cntn, cnte)


def _tc_matmul(x, w, crow):
    m, kdim = x.shape
    blk = 1000

    def body(x_ref, w_ref, c_ref, o_ref):
        o_ref[...] = (jnp.dot(x_ref[...], w_ref[...],
                              preferred_element_type=jnp.float32)
                      + c_ref[...])

    return pl.pallas_call(
        body,
        grid=(m // blk,),
        in_specs=[pl.BlockSpec((blk, kdim), lambda i: (i, 0)),
                  pl.BlockSpec(w.shape, lambda i: (0, 0)),
                  pl.BlockSpec((1, _HID), lambda i: (0, 0))],
        out_specs=pl.BlockSpec((blk, _HID), lambda i: (i, 0)),
        out_shape=jax.ShapeDtypeStruct((m, _HID), jnp.float32),
    )(x, w, crow)


def _tc_combine(p, scale, leaky=False):
    r = p.shape[1]
    blk = 1000

    def body(p_ref, s_ref, o_ref):
        pp = p_ref[...]
        v = (pp[0] + pp[1]) * s_ref[...]
        if leaky:
            v = jnp.where(v >= 0, v, 0.2 * v)
        o_ref[...] = v

    return pl.pallas_call(
        body,
        grid=(r // blk,),
        in_specs=[pl.BlockSpec((2, blk, 128), lambda i: (0, i, 0)),
                  pl.BlockSpec((blk, 1), lambda i: (i, 0))],
        out_specs=pl.BlockSpec((blk, 128), lambda i: (i, 0)),
        out_shape=jax.ShapeDtypeStruct((r, 128), jnp.float32),
    )(p, scale)


def _tc_gate(p, inv_edge, wa, ba2):
    blk = 1000

    def body(p_ref, s_ref, wa_ref, ba_ref, o_ref):
        pp = p_ref[...]
        ef = (pp[0] + pp[1]) * s_ref[...]
        logit = (jnp.dot(ef, wa_ref[...], preferred_element_type=jnp.float32)
                 + ba_ref[...])
        gate = 1.0 / (1.0 + jnp.exp(-logit))
        o_ref[...] = ef * gate

    return pl.pallas_call(
        body,
        grid=(_E // blk,),
        in_specs=[pl.BlockSpec((2, blk, 128), lambda i: (0, i, 0)),
                  pl.BlockSpec((blk, 1), lambda i: (i, 0)),
                  pl.BlockSpec((_HID, 1), lambda i: (0, 0)),
                  pl.BlockSpec((1, 1), lambda i: (0, 0))],
        out_specs=pl.BlockSpec((blk, 128), lambda i: (i, 0)),
        out_shape=jax.ShapeDtypeStruct((_E, 128), jnp.float32),
    )(p, inv_edge, wa, ba2)


def kernel(X, node_idx, edge_idx, visit_times, Wt, bt, theta1, b1, Wa1, ba1,
           theta2, b2, Wa2, ba2):
    pass_e = _sc_scatter_pass(_E)
    pass_n = _sc_scatter_pass(_N)

    ones_rows = jnp.ones((8, 128), jnp.float32)
    zidx = jnp.zeros((_NNZ,), jnp.int32)
    cntn = pass_n(ones_rows, zidx, node_idx)[:, :, :1]
    cnte = pass_e(ones_rows, zidx, edge_idx)[:, :, :1]
    const1, scale_e, inv_edge, inv_deg = _tc_prep(
        visit_times.reshape(_E, 1), Wt, bt.reshape(1, _T), theta1[_D:],
        b1.reshape(1, _HID), cntn, cnte)

    def one_layer(xin, w, crow, wa, ba):
        y = _tc_matmul(xin, w, crow)
        pe = pass_e(y, node_idx, edge_idx)
        e = _tc_combine(pe, scale_e)
        pn = pass_n(e, edge_idx, node_idx)
        x1 = _tc_combine(pn, inv_deg)
        pe2 = pass_e(x1, node_idx, edge_idx)
        msg = _tc_gate(pe2, inv_edge, wa, ba.reshape(1, 1))
        pn2 = pass_n(msg, edge_idx, node_idx)
        return _tc_combine(pn2, inv_deg, leaky=True)

    h = one_layer(X, theta1[:_D], const1, Wa1, ba1)
    h = one_layer(h, theta2, b2.reshape(1, _HID), Wa2, ba2)
    return h


# speedup vs baseline: 7.8475x; 7.8475x over previous
import jax
import jax.numpy as jnp
from jax import lax
from jax.experimental import pallas as pl
from jax.experimental.pallas import tpu as pltpu
from jax.experimental.pallas import tpu_sc as plsc

_N = 10000
_E = 5000
_NNZ = 320000
_D = 128
_HID = 128
_T = 16

_K = 128
_NCHUNK = _NNZ // _K
_NC = 2
_NS = 16
_NW = _NC * _NS
_ZROWS = 200


def _sc_mesh():
    return plsc.VectorSubcoreMesh(core_axis_name="c", subcore_axis_name="s")


def _sc_scatter_pass(r_out):
    nz = r_out // _ZROWS

    def body(in_hbm, si_hbm, di_hbm, out_hbm, si_v, di_v, rows_v, zero_v,
             acc, sem):
        c = lax.axis_index("c")
        s = lax.axis_index("s")
        w = s * _NC + c

        z16 = jnp.zeros((16,), jnp.float32)

        def zrow(i, carry):
            for j in range(128 // 16):
                zero_v[i, pl.ds(j * 16, 16)] = z16
            return carry

        lax.fori_loop(0, _ZROWS, zrow, 0)

        def zchunk(i, carry):
            t = s + i * _NS
            pltpu.sync_copy(zero_v, acc.at[pl.ds(t * _ZROWS, _ZROWS)])
            return carry

        lax.fori_loop(0, (nz - s + _NS - 1) // _NS, zchunk, 0)
        plsc.subcore_barrier()

        def chunk(i, carry):
            t = w + i * _NW
            base = t * _K
            pltpu.sync_copy(si_hbm.at[pl.ds(base, _K)], si_v)
            pltpu.sync_copy(di_hbm.at[pl.ds(base, _K)], di_v)
            pltpu.async_copy(in_hbm.at[si_v], rows_v, sem).wait()
            pltpu.sync_copy(rows_v, acc.at[di_v], add=True)
            return carry

        lax.fori_loop(0, (_NCHUNK - w + _NW - 1) // _NW, chunk, 0)
        plsc.subcore_barrier()

        def ochunk(i, carry):
            t = s + i * _NS
            pltpu.sync_copy(acc.at[pl.ds(t * _ZROWS, _ZROWS)],
                            out_hbm.at[c, pl.ds(t * _ZROWS, _ZROWS)])
            return carry

        lax.fori_loop(0, (nz - s + _NS - 1) // _NS, ochunk, 0)

    return pl.kernel(
        body,
        out_type=jax.ShapeDtypeStruct((_NC, r_out, 128), jnp.float32),
        mesh=_sc_mesh(),
        scratch_types=[
            pltpu.VMEM((_K,), jnp.int32),
            pltpu.VMEM((_K,), jnp.int32),
            pltpu.VMEM((_K, 128), jnp.float32),
            pltpu.VMEM((_ZROWS, 128), jnp.float32),
            pltpu.VMEM_SHARED((r_out, 128), jnp.float32),
            pltpu.SemaphoreType.DMA,
        ],
    )


def _sc_count_kernel():
    nzn = _N // _ZROWS
    nze = _E // _ZROWS

    def body(ni_hbm, ei_hbm, outn_hbm, oute_hbm, ni_v, ei_v, ones_v, zero_v,
             accn, acce):
        c = lax.axis_index("c")
        s = lax.axis_index("s")
        w = s * _NC + c

        one16 = jnp.ones((16,), jnp.float32)
        z16 = jnp.zeros((16,), jnp.float32)

        def fill(i, carry):
            ones_v[i, :] = one16
            return carry

        lax.fori_loop(0, _K, fill, 0)

        def zrow(i, carry):
            zero_v[i, :] = z16
            return carry

        lax.fori_loop(0, _ZROWS, zrow, 0)

        def zn(i, carry):
            t = s + i * _NS
            pltpu.sync_copy(zero_v, accn.at[pl.ds(t * _ZROWS, _ZROWS)])
            return carry

        lax.fori_loop(0, (nzn - s + _NS - 1) // _NS, zn, 0)

        def ze(i, carry):
            t = s + i * _NS
            pltpu.sync_copy(zero_v, acce.at[pl.ds(t * _ZROWS, _ZROWS)])
            return carry

        lax.fori_loop(0, (nze - s + _NS - 1) // _NS, ze, 0)
        plsc.subcore_barrier()

        def chunk(i, carry):
            t = w + i * _NW
            base = t * _K
            pltpu.sync_copy(ni_hbm.at[pl.ds(base, _K)], ni_v)
            pltpu.sync_copy(ei_hbm.at[pl.ds(base, _K)], ei_v)
            pltpu.sync_copy(ones_v, accn.at[ni_v], add=True)
            pltpu.sync_copy(ones_v, acce.at[ei_v], add=True)
            return carry

        lax.fori_loop(0, (_NCHUNK - w + _NW - 1) // _NW, chunk, 0)
        plsc.subcore_barrier()

        def on(i, carry):
            t = s + i * _NS
            pltpu.sync_copy(accn.at[pl.ds(t * _ZROWS, _ZROWS)],
                            outn_hbm.at[c, pl.ds(t * _ZROWS, _ZROWS)])
            return carry

        lax.fori_loop(0, (nzn - s + _NS - 1) // _NS, on, 0)

        def oe(i, carry):
            t = s + i * _NS
            pltpu.sync_copy(acce.at[pl.ds(t * _ZROWS, _ZROWS)],
                            oute_hbm.at[c, pl.ds(t * _ZROWS, _ZROWS)])
            return carry

        lax.fori_loop(0, (nze - s + _NS - 1) // _NS, oe, 0)

    return pl.kernel(
        body,
        out_type=(jax.ShapeDtypeStruct((_NC, _N, 16), jnp.float32),
                  jax.ShapeDtypeStruct((_NC, _E, 16), jnp.float32)),
        mesh=_sc_mesh(),
        scratch_types=[
            pltpu.VMEM((_K,), jnp.int32),
            pltpu.VMEM((_K,), jnp.int32),
            pltpu.VMEM((_K, 16), jnp.float32),
            pltpu.VMEM((_ZROWS, 16), jnp.float32),
            pltpu.VMEM_SHARED((_N, 16), jnp.float32),
            pltpu.VMEM_SHARED((_E, 16), jnp.float32),
        ],
    )


def _tc_prep(vt2, wt, bt2, th1t, b12, cntn, cnte):
    def body(vt_ref, wt_ref, bt_ref, tht_ref, b1_ref, cn_ref, ce_ref,
             c1_ref, se_ref, ie_ref, idg_ref):
        vt = vt_ref[...]
        vmax = jnp.max(vt)
        vmean = jnp.sum(vt) * (1.0 / _E)
        tf = vmean * wt_ref[...] + bt_ref[...]
        c1_ref[...] = (jnp.dot(tf, tht_ref[...],
                               preferred_element_type=jnp.float32)
                       + b1_ref[...])
        decay = jnp.exp((vt - vmax) * (1.0 / 365.0))
        cn = cn_ref[...]
        ce = ce_ref[...]
        esize = jnp.maximum(ce[0] + ce[1], 1.0)
        ndeg = jnp.maximum(cn[0] + cn[1], 1.0)
        se_ref[...] = decay / esize
        ie_ref[...] = 1.0 / esize
        idg_ref[...] = 1.0 / ndeg

    return pl.pallas_call(
        body,
        out_shape=(jax.ShapeDtypeStruct((1, _HID), jnp.float32),
                   jax.ShapeDtypeStruct((_E, 1), jnp.float32),
                   jax.ShapeDtypeStruct((_E, 1), jnp.float32),
                   jax.ShapeDtypeStruct((_N, 1), jnp.float32)),
    )(vt2, wt, bt2, th1t, b12, cntn, cnte)


def _tc_matmul(x, w, crow):
    m, kdim = x.shape
    blk = 1000

    def body(x_ref, w_ref, c_ref, o_ref):
        o_ref[...] = (jnp.dot(x_ref[...], w_ref[...],
                              preferred_element_type=jnp.float32)
                      + c_ref[...])

    return pl.pallas_call(
        body,
        grid=(m // blk,),
        in_specs=[pl.BlockSpec((blk, kdim), lambda i: (i, 0)),
                  pl.BlockSpec(w.shape, lambda i: (0, 0)),
                  pl.BlockSpec((1, _HID), lambda i: (0, 0))],
        out_specs=pl.BlockSpec((blk, _HID), lambda i: (i, 0)),
        out_shape=jax.ShapeDtypeStruct((m, _HID), jnp.float32),
    )(x, w, crow)


def _tc_combine(p, scale, leaky=False):
    r = p.shape[1]
    blk = 1000

    def body(p_ref, s_ref, o_ref):
        pp = p_ref[...]
        v = (pp[0] + pp[1]) * s_ref[...]
        if leaky:
            v = jnp.where(v >= 0, v, 0.2 * v)
        o_ref[...] = v

    return pl.pallas_call(
        body,
        grid=(r // blk,),
        in_specs=[pl.BlockSpec((2, blk, 128), lambda i: (0, i, 0)),
                  pl.BlockSpec((blk, 1), lambda i: (i, 0))],
        out_specs=pl.BlockSpec((blk, 128), lambda i: (i, 0)),
        out_shape=jax.ShapeDtypeStruct((r, 128), jnp.float32),
    )(p, scale)


def _tc_gate(p, inv_edge, wa, ba2):
    blk = 1000

    def body(p_ref, s_ref, wa_ref, ba_ref, o_ref):
        pp = p_ref[...]
        ef = (pp[0] + pp[1]) * s_ref[...]
        logit = (jnp.dot(ef, wa_ref[...], preferred_element_type=jnp.float32)
                 + ba_ref[...])
        gate = 1.0 / (1.0 + jnp.exp(-logit))
        o_ref[...] = ef * gate

    return pl.pallas_call(
        body,
        grid=(_E // blk,),
        in_specs=[pl.BlockSpec((2, blk, 128), lambda i: (0, i, 0)),
                  pl.BlockSpec((blk, 1), lambda i: (i, 0)),
                  pl.BlockSpec((_HID, 1), lambda i: (0, 0)),
                  pl.BlockSpec((1, 1), lambda i: (0, 0))],
        out_specs=pl.BlockSpec((blk, 128), lambda i: (i, 0)),
        out_shape=jax.ShapeDtypeStruct((_E, 128), jnp.float32),
    )(p, inv_edge, wa, ba2)


def kernel(X, node_idx, edge_idx, visit_times, Wt, bt, theta1, b1, Wa1, ba1,
           theta2, b2, Wa2, ba2):
    pass_e = _sc_scatter_pass(_E)
    pass_n = _sc_scatter_pass(_N)

    cntn, cnte = _sc_count_kernel()(node_idx, edge_idx)
    cntn = cntn[:, :, :1]
    cnte = cnte[:, :, :1]
    const1, scale_e, inv_edge, inv_deg = _tc_prep(
        visit_times.reshape(_E, 1), Wt, bt.reshape(1, _T), theta1[_D:],
        b1.reshape(1, _HID), cntn, cnte)

    def one_layer(xin, w, crow, wa, ba):
        y = _tc_matmul(xin, w, crow)
        pe = pass_e(y, node_idx, edge_idx)
        e = _tc_combine(pe, scale_e)
        pn = pass_n(e, edge_idx, node_idx)
        x1 = _tc_combine(pn, inv_deg)
        pe2 = pass_e(x1, node_idx, edge_idx)
        msg = _tc_gate(pe2, inv_edge, wa, ba.reshape(1, 1))
        pn2 = pass_n(msg, edge_idx, node_idx)
        return _tc_combine(pn2, inv_deg, leaky=True)

    h = one_layer(X, theta1[:_D], const1, Wa1, ba1)
    h = one_layer(h, theta2, b2.reshape(1, _HID), Wa2, ba2)
    return h


# speedup vs baseline: 14.2793x; 1.8196x over previous
import jax
import jax.numpy as jnp
from jax import lax
from jax.experimental import pallas as pl
from jax.experimental.pallas import tpu as pltpu
from jax.experimental.pallas import tpu_sc as plsc

_N = 10000
_E = 5000
_NNZ = 320000
_D = 128
_HID = 128
_T = 16

_K = 128
_NCHUNK = _NNZ // _K
_NC = 2
_NS = 16
_NW = _NC * _NS
_ZROWS = 40


def _sc_mesh():
    return plsc.VectorSubcoreMesh(core_axis_name="c", subcore_axis_name="s")


_CPW = _NCHUNK // _NW
_XTRA = _NCHUNK - _CPW * _NW
_BLK = 40
_BLK2 = _CPW - _BLK


def _sc_scatter_pass(r_out):
    nz = r_out // _ZROWS

    def body(in_hbm, si_hbm, di_hbm, out_hbm, si_v, di_v, ra, rb, zero_v,
             acc, sema, semb):
        c = lax.axis_index("c")
        s = lax.axis_index("s")
        w = s * _NC + c
        start = w * _CPW + jnp.minimum(w, _XTRA)
        extra = w < _XTRA

        z16 = jnp.zeros((16,), jnp.float32)

        def zrow(i, carry):
            for j in range(128 // 16):
                zero_v[i, pl.ds(j * 16, 16)] = z16
            return carry

        lax.fori_loop(0, _ZROWS, zrow, 0)

        def zchunk(i, carry):
            t = s + i * _NS
            pltpu.sync_copy(zero_v, acc.at[pl.ds(t * _ZROWS, _ZROWS)])
            return carry

        lax.fori_loop(0, (nz - s + _NS - 1) // _NS, zchunk, 0)

        plsc.subcore_barrier()

        def run_block(n, tail):
            pltpu.async_copy(in_hbm.at[si_v.at[0, 0]], ra, sema)

            def step(i2, carry):
                a = 2 * i2
                b = a + 1
                pltpu.async_copy(in_hbm.at[si_v.at[b, 0]], rb, semb)
                pltpu.make_async_copy(in_hbm.at[si_v.at[a, 0]], ra,
                                      sema).wait()
                pltpu.sync_copy(ra, acc.at[di_v.at[a, 0]], add=True)
                nxt = a + 2
                more = nxt < n if tail is None else jnp.logical_or(nxt < n,
                                                                   tail)

                @pl.when(more)
                def _():
                    pltpu.async_copy(in_hbm.at[si_v.at[nxt, 0]], ra, sema)

                pltpu.make_async_copy(in_hbm.at[si_v.at[b, 0]], rb,
                                      semb).wait()
                pltpu.sync_copy(rb, acc.at[di_v.at[b, 0]], add=True)
                return carry

            lax.fori_loop(0, n // 2, step, 0)

            if tail is not None:
                @pl.when(tail)
                def _():
                    pltpu.make_async_copy(in_hbm.at[si_v.at[n, 0]], ra,
                                          sema).wait()
                    pltpu.sync_copy(ra, acc.at[di_v.at[n, 0]], add=True)

        pltpu.sync_copy(si_hbm.at[pl.ds(start, _BLK)], si_v.at[pl.ds(0, _BLK)])
        pltpu.sync_copy(di_hbm.at[pl.ds(start, _BLK)], di_v.at[pl.ds(0, _BLK)])
        run_block(_BLK, None)

        pltpu.sync_copy(si_hbm.at[pl.ds(start + _BLK, _BLK2)],
                        si_v.at[pl.ds(0, _BLK2)])
        pltpu.sync_copy(di_hbm.at[pl.ds(start + _BLK, _BLK2)],
                        di_v.at[pl.ds(0, _BLK2)])

        @pl.when(extra)
        def _():
            pltpu.sync_copy(si_hbm.at[pl.ds(start + _CPW, 1)],
                            si_v.at[pl.ds(_BLK2, 1)])
            pltpu.sync_copy(di_hbm.at[pl.ds(start + _CPW, 1)],
                            di_v.at[pl.ds(_BLK2, 1)])

        run_block(_BLK2, extra)

        plsc.subcore_barrier()

        def ochunk(i, carry):
            t = s + i * _NS
            pltpu.sync_copy(acc.at[pl.ds(t * _ZROWS, _ZROWS)],
                            out_hbm.at[c, pl.ds(t * _ZROWS, _ZROWS)])
            return carry

        lax.fori_loop(0, (nz - s + _NS - 1) // _NS, ochunk, 0)

    return pl.kernel(
        body,
        out_type=jax.ShapeDtypeStruct((_NC, r_out, 128), jnp.float32),
        mesh=_sc_mesh(),
        scratch_types=[
            pltpu.VMEM((_BLK, 1, _K), jnp.int32),
            pltpu.VMEM((_BLK, 1, _K), jnp.int32),
            pltpu.VMEM((_K, 128), jnp.float32),
            pltpu.VMEM((_K, 128), jnp.float32),
            pltpu.VMEM((_ZROWS, 128), jnp.float32),
            pltpu.VMEM_SHARED((r_out, 128), jnp.float32),
            pltpu.SemaphoreType.DMA,
            pltpu.SemaphoreType.DMA,
        ],
    )


def _sc_count_kernel():
    nzn = _N // _ZROWS
    nze = _E // _ZROWS

    def body(ni_hbm, ei_hbm, outn_hbm, oute_hbm, ni_v, ei_v, ones_v, zero_v,
             accn, acce):
        c = lax.axis_index("c")
        s = lax.axis_index("s")
        w = s * _NC + c

        one16 = jnp.ones((16,), jnp.float32)
        z16 = jnp.zeros((16,), jnp.float32)

        def fill(i, carry):
            ones_v[i, :] = one16
            return carry

        lax.fori_loop(0, _K, fill, 0)

        def zrow(i, carry):
            zero_v[i, :] = z16
            return carry

        lax.fori_loop(0, _ZROWS, zrow, 0)

        def zn(i, carry):
            t = s + i * _NS
            pltpu.sync_copy(zero_v, accn.at[pl.ds(t * _ZROWS, _ZROWS)])
            return carry

        lax.fori_loop(0, (nzn - s + _NS - 1) // _NS, zn, 0)

        def ze(i, carry):
            t = s + i * _NS
            pltpu.sync_copy(zero_v, acce.at[pl.ds(t * _ZROWS, _ZROWS)])
            return carry

        lax.fori_loop(0, (nze - s + _NS - 1) // _NS, ze, 0)
        plsc.subcore_barrier()

        def chunk(i, carry):
            t = w + i * _NW
            base = t * _K
            pltpu.sync_copy(ni_hbm.at[pl.ds(base, _K)], ni_v)
            pltpu.sync_copy(ei_hbm.at[pl.ds(base, _K)], ei_v)
            pltpu.sync_copy(ones_v, accn.at[ni_v], add=True)
            pltpu.sync_copy(ones_v, acce.at[ei_v], add=True)
            return carry

        lax.fori_loop(0, (_NCHUNK - w + _NW - 1) // _NW, chunk, 0)
        plsc.subcore_barrier()

        def on(i, carry):
            t = s + i * _NS
            pltpu.sync_copy(accn.at[pl.ds(t * _ZROWS, _ZROWS)],
                            outn_hbm.at[c, pl.ds(t * _ZROWS, _ZROWS)])
            return carry

        lax.fori_loop(0, (nzn - s + _NS - 1) // _NS, on, 0)

        def oe(i, carry):
            t = s + i * _NS
            pltpu.sync_copy(acce.at[pl.ds(t * _ZROWS, _ZROWS)],
                            oute_hbm.at[c, pl.ds(t * _ZROWS, _ZROWS)])
            return carry

        lax.fori_loop(0, (nze - s + _NS - 1) // _NS, oe, 0)

    return pl.kernel(
        body,
        out_type=(jax.ShapeDtypeStruct((_NC, _N, 16), jnp.float32),
                  jax.ShapeDtypeStruct((_NC, _E, 16), jnp.float32)),
        mesh=_sc_mesh(),
        scratch_types=[
            pltpu.VMEM((_K,), jnp.int32),
            pltpu.VMEM((_K,), jnp.int32),
            pltpu.VMEM((_K, 16), jnp.float32),
            pltpu.VMEM((_ZROWS, 16), jnp.float32),
            pltpu.VMEM_SHARED((_N, 16), jnp.float32),
            pltpu.VMEM_SHARED((_E, 16), jnp.float32),
        ],
    )


def _tc_prep(vt2, wt, bt2, th1t, b12, cntn, cnte):
    def body(vt_ref, wt_ref, bt_ref, tht_ref, b1_ref, cn_ref, ce_ref,
             c1_ref, se_ref, ie_ref, idg_ref):
        vt = vt_ref[...]
        vmax = jnp.max(vt)
        vmean = jnp.sum(vt) * (1.0 / _E)
        tf = vmean * wt_ref[...] + bt_ref[...]
        c1_ref[...] = (jnp.dot(tf, tht_ref[...],
                               preferred_element_type=jnp.float32)
                       + b1_ref[...])
        decay = jnp.exp((vt - vmax) * (1.0 / 365.0))
        cn = cn_ref[...]
        ce = ce_ref[...]
        esize = jnp.maximum(ce[0] + ce[1], 1.0)
        ndeg = jnp.maximum(cn[0] + cn[1], 1.0)
        se_ref[...] = decay / esize
        ie_ref[...] = 1.0 / esize
        idg_ref[...] = 1.0 / ndeg

    return pl.pallas_call(
        body,
        out_shape=(jax.ShapeDtypeStruct((1, _HID), jnp.float32),
                   jax.ShapeDtypeStruct((_E, 1), jnp.float32),
                   jax.ShapeDtypeStruct((_E, 1), jnp.float32),
                   jax.ShapeDtypeStruct((_N, 1), jnp.float32)),
    )(vt2, wt, bt2, th1t, b12, cntn, cnte)


def _tc_matmul(x, w, crow):
    m, kdim = x.shape
    blk = 1000

    def body(x_ref, w_ref, c_ref, o_ref):
        o_ref[...] = (jnp.dot(x_ref[...], w_ref[...],
                              preferred_element_type=jnp.float32)
                      + c_ref[...])

    return pl.pallas_call(
        body,
        grid=(m // blk,),
        in_specs=[pl.BlockSpec((blk, kdim), lambda i: (i, 0)),
                  pl.BlockSpec(w.shape, lambda i: (0, 0)),
                  pl.BlockSpec((1, _HID), lambda i: (0, 0))],
        out_specs=pl.BlockSpec((blk, _HID), lambda i: (i, 0)),
        out_shape=jax.ShapeDtypeStruct((m, _HID), jnp.float32),
    )(x, w, crow)


def _tc_combine(p, scale, leaky=False):
    r = p.shape[1]
    blk = 1000

    def body(p_ref, s_ref, o_ref):
        pp = p_ref[...]
        v = (pp[0] + pp[1]) * s_ref[...]
        if leaky:
            v = jnp.where(v >= 0, v, 0.2 * v)
        o_ref[...] = v

    return pl.pallas_call(
        body,
        grid=(r // blk,),
        in_specs=[pl.BlockSpec((2, blk, 128), lambda i: (0, i, 0)),
                  pl.BlockSpec((blk, 1), lambda i: (i, 0))],
        out_specs=pl.BlockSpec((blk, 128), lambda i: (i, 0)),
        out_shape=jax.ShapeDtypeStruct((r, 128), jnp.float32),
    )(p, scale)


def _tc_gate(p, inv_edge, wa, ba2):
    blk = 1000

    def body(p_ref, s_ref, wa_ref, ba_ref, o_ref):
        pp = p_ref[...]
        ef = (pp[0] + pp[1]) * s_ref[...]
        logit = (jnp.dot(ef, wa_ref[...], preferred_element_type=jnp.float32)
                 + ba_ref[...])
        gate = 1.0 / (1.0 + jnp.exp(-logit))
        o_ref[...] = ef * gate

    return pl.pallas_call(
        body,
        grid=(_E // blk,),
        in_specs=[pl.BlockSpec((2, blk, 128), lambda i: (0, i, 0)),
                  pl.BlockSpec((blk, 1), lambda i: (i, 0)),
                  pl.BlockSpec((_HID, 1), lambda i: (0, 0)),
                  pl.BlockSpec((1, 1), lambda i: (0, 0))],
        out_specs=pl.BlockSpec((blk, 128), lambda i: (i, 0)),
        out_shape=jax.ShapeDtypeStruct((_E, 128), jnp.float32),
    )(p, inv_edge, wa, ba2)


def kernel(X, node_idx, edge_idx, visit_times, Wt, bt, theta1, b1, Wa1, ba1,
           theta2, b2, Wa2, ba2):
    pass_e = _sc_scatter_pass(_E)
    pass_n = _sc_scatter_pass(_N)

    cntn, cnte = _sc_count_kernel()(node_idx, edge_idx)
    cntn = cntn[:, :, :1]
    cnte = cnte[:, :, :1]

    ni2 = node_idx.reshape(_NCHUNK, 1, _K)
    ei2 = edge_idx.reshape(_NCHUNK, 1, _K)
    const1, scale_e, inv_edge, inv_deg = _tc_prep(
        visit_times.reshape(_E, 1), Wt, bt.reshape(1, _T), theta1[_D:],
        b1.reshape(1, _HID), cntn, cnte)

    def one_layer(xin, w, crow, wa, ba):
        y = _tc_matmul(xin, w, crow)
        pe = pass_e(y, ni2, ei2)
        e = _tc_combine(pe, scale_e)
        pn = pass_n(e, ei2, ni2)
        x1 = _tc_combine(pn, inv_deg)
        pe2 = pass_e(x1, ni2, ei2)
        msg = _tc_gate(pe2, inv_edge, wa, ba.reshape(1, 1))
        pn2 = pass_n(msg, ei2, ni2)
        return _tc_combine(pn2, inv_deg, leaky=True)

    h = one_layer(X, theta1[:_D], const1, Wa1, ba1)
    h = one_layer(h, theta2, b2.reshape(1, _HID), Wa2, ba2)
    return h
